# baseline (speedup 1.0000x reference)
import functools

import jax
import jax.numpy as jnp
from jax import lax
from jax.experimental import pallas as pl
from jax.experimental.pallas import tpu as pltpu
from jax.experimental.pallas import tpu_sc as plsc

B = 16384
IN = 118
K = 1024
D = 64
NG = 2
COMMIT = 0.25

BBE = 4096
NBE = B // BBE
BBD = 4096
NBD = B // BBD

NC = 2
NS = 16
NW = NC * NS
BPW = B // NW
L = 16


def _encode_body(xt_ref, w1, w2, w3, w4t, et_ref, cst_ref,
                 zet_ref, idx_ref):
    cst = cst_ref[...]
    esq = cst[0:K]
    b1 = cst[K:K + 60]
    b2 = cst[K + 64:K + 64 + 30]
    b3 = cst[K + 128:K + 128 + 10]
    b4 = cst[K + 192:K + 192 + D]
    xt = xt_ref[...]
    h = jnp.tanh(jnp.dot(w1[...], xt) + b1)
    h = jnp.tanh(jnp.dot(w2[...], h) + b2)
    h = jnp.tanh(jnp.dot(w3[...], h) + b3)
    zet = lax.dot_general(w4t[...], h, (((0,), (0,)), ((), ()))) + b4
    zet_ref[...] = zet
    m2 = lax.dot_general(et_ref[...], -2.0 * zet,
                         (((0,), (0,)), ((), ())))
    zsq = jnp.sum(zet * zet, axis=0, keepdims=True)
    dist = (zsq + esq) + m2
    dmin = jnp.min(dist, axis=0, keepdims=True)
    iota = lax.broadcasted_iota(jnp.int32, dist.shape, 0)
    idx = jnp.min(jnp.where(dist == dmin, iota, jnp.int32(K)), axis=0,
                  keepdims=True)
    idx_ref[...] = idx


def _encode(xt, w1, w2, w3, w4, et, cst):
    full = lambda *s: pl.BlockSpec(s, lambda i: (0,) * len(s))
    return pl.pallas_call(
        _encode_body,
        grid=(NBE,),
        in_specs=[
            pl.BlockSpec((IN, BBE), lambda i: (0, i)),
            full(60, IN),
            full(30, 60),
            full(10, 30),
            full(10, D),
            full(D, K), full(K + 256, 1),
        ],
        out_specs=[
            pl.BlockSpec((D, BBE), lambda i: (0, i)),
            pl.BlockSpec((1, BBE), lambda i: (0, i)),
        ],
        out_shape=[
            jax.ShapeDtypeStruct((D, B), jnp.float32),
            jax.ShapeDtypeStruct((1, B), jnp.int32),
        ],
    )(xt, w1, w2, w3, w4, et, cst)


_sc_gather_impl = None


def _build_sc_gather():
    mesh = plsc.VectorSubcoreMesh(core_axis_name="c", subcore_axis_name="s",
                                  num_cores=NC, num_subcores=NS)
    DGN = 4
    BGN = NW // DGN
    DPG = D // DGN
    BPG = B // BGN
    NGRP = BPG // L

    @functools.partial(
        pl.kernel,
        mesh=mesh,
        compiler_params=pltpu.CompilerParams(needs_layout_passes=False,
                                             skip_device_barrier=True),
        out_type=jax.ShapeDtypeStruct((D, B), jnp.float32),
        scratch_types=[
            pltpu.VMEM((DPG, K), jnp.float32),
            pltpu.VMEM((BPG,), jnp.int32),
            pltpu.VMEM((DPG, BPG), jnp.float32),
        ],
    )
    def body(et_hbm, idx_hbm, out_hbm, et_v, idx_v, zqt_v):
        wid = lax.axis_index("s") * NC + lax.axis_index("c")
        dg = wid // BGN
        bg = wid % BGN
        pltpu.sync_copy(et_hbm.at[pl.ds(dg * DPG, DPG)], et_v)
        pltpu.sync_copy(idx_hbm.at[pl.ds(bg * BPG, BPG)], idx_v)

        def grp_body(g, carry):
            idxv = idx_v[pl.ds(g * L, L)]
            for dl in range(DPG):
                dsplat = jnp.full((L,), dl, jnp.int32)
                col = plsc.load_gather(et_v, [dsplat, idxv])
                zqt_v[dl, pl.ds(g * L, L)] = col
            return carry

        lax.fori_loop(0, NGRP, grp_body, 0, unroll=4)
        pltpu.sync_copy(
            zqt_v, out_hbm.at[pl.ds(dg * DPG, DPG), pl.ds(bg * BPG, BPG)])

    return body


def _sc_gather(et_flat, idx):
    global _sc_gather_impl
    if _sc_gather_impl is None:
        _sc_gather_impl = _build_sc_gather()
    return _sc_gather_impl(et_flat, idx)


def _decode_body(xt_ref, zet_ref, zqt_in_ref, w5, w6t, w7t, w8t, w9, w10,
                 cst_ref, zqt_ref, xht_ref, zaugt_ref, gt_ref, vq_ref):
    cst = cst_ref[...]
    b5 = cst[0:10]
    b6 = cst[64:64 + 30]
    b7 = cst[128:128 + 60]
    b8 = cst[192:192 + IN]
    b9 = cst[320:320 + 10]
    b10 = cst[384:384 + NG]
    zqt = zqt_in_ref[...]
    zqt_ref[...] = zqt
    tdot = lambda a, b: lax.dot_general(a, b, (((0,), (0,)), ((), ())))
    h = jnp.tanh(jnp.dot(w5[...], zqt) + b5)
    h = jnp.tanh(tdot(w6t[...], h) + b6)
    h = jnp.tanh(tdot(w7t[...], h) + b7)
    xht = tdot(w8t[...], h) + b8
    xht_ref[...] = xht
    xt = xt_ref[...]
    diff = xt - xht
    dn = jnp.sqrt(jnp.sum(diff * diff, axis=0, keepdims=True))
    xn = jnp.sqrt(jnp.sum(xt * xt, axis=0, keepdims=True))
    xhn = jnp.sqrt(jnp.sum(xht * xht, axis=0, keepdims=True))
    rec1 = dn / (xn + 1e-12)
    rec2 = jnp.sum(xt * xht, axis=0, keepdims=True) / (
        jnp.maximum(xn, 1e-8) * jnp.maximum(xhn, 1e-8))
    zaugt = jnp.concatenate([zqt, rec1, rec2], axis=0)
    zaugt_ref[...] = zaugt
    h9 = jnp.tanh(jnp.dot(w9[...], zaugt) + b9)
    logits = jnp.dot(w10[...], h9) + b10
    lmax = jnp.max(logits, axis=0, keepdims=True)
    e = jnp.exp(logits - lmax)
    gt_ref[...] = e / jnp.sum(e, axis=0, keepdims=True)
    d = zqt - zet_ref[...]
    s = jnp.sum(d * d)
    i = pl.program_id(0)

    @pl.when(i == 0)
    def _():
        vq_ref[0, 0, 0] = s

    @pl.when(i > 0)
    def _():
        vq_ref[0, 0, 0] += s

    @pl.when(i == NBD - 1)
    def _():
        m = vq_ref[0, 0, 0] / (B * D)
        vq_ref[0, 0, 0] = m + COMMIT * m


def _decode(xt, zet, zqt, w5, w6, w7, w8, w9, w10, cst):
    full = lambda *s: pl.BlockSpec(s, lambda i: (0,) * len(s))
    return pl.pallas_call(
        _decode_body,
        grid=(NBD,),
        in_specs=[
            pl.BlockSpec((IN, BBD), lambda i: (0, i)),
            pl.BlockSpec((D, BBD), lambda i: (0, i)),
            pl.BlockSpec((D, BBD), lambda i: (0, i)),
            full(10, D),
            full(10, 30),
            full(30, 60),
            full(60, IN),
            full(10, D + 2),
            full(NG, 10),
            full(386, 1),
        ],
        out_specs=[
            pl.BlockSpec((D, BBD), lambda i: (0, i)),
            pl.BlockSpec((IN, BBD), lambda i: (0, i)),
            pl.BlockSpec((D + 2, BBD), lambda i: (0, i)),
            pl.BlockSpec((NG, BBD), lambda i: (0, i)),
            pl.BlockSpec((1, 1, 1), lambda i: (0, 0, 0),
                         memory_space=pltpu.SMEM),
        ],
        out_shape=[
            jax.ShapeDtypeStruct((D, B), jnp.float32),
            jax.ShapeDtypeStruct((IN, B), jnp.float32),
            jax.ShapeDtypeStruct((D + 2, B), jnp.float32),
            jax.ShapeDtypeStruct((NG, B), jnp.float32),
            jax.ShapeDtypeStruct((1, 1, 1), jnp.float32),
        ],
    )(xt, zet, zqt, w5, w6, w7, w8, w9, w10, cst)


def kernel(x, params):
    p = params
    xt = x.T
    et = p['codebook'].T
    esq = jnp.sum(p['codebook'] ** 2, axis=1)

    enc_cst = jnp.concatenate([
        esq,
        jnp.pad(p['b1'], (0, 4)), jnp.pad(p['b2'], (0, 34)),
        jnp.pad(p['b3'], (0, 54)), p['b4'],
    ]).reshape(K + 256, 1)
    dec_cst = jnp.concatenate([
        jnp.pad(p['b5'], (0, 54)), jnp.pad(p['b6'], (0, 34)),
        jnp.pad(p['b7'], (0, 4)), jnp.pad(p['b8'], (0, 10)),
        jnp.pad(p['b9'], (0, 54)), p['b10'],
    ]).reshape(386, 1)

    zet, idx2 = _encode(xt, p['W1'], p['W2'], p['W3'], p['W4'].T, et,
                        enc_cst)

    zqt_sc = _sc_gather(et, idx2.reshape(B))

    zqt, xht, zaugt, gt, vq_out = _decode(
        xt, zet, zqt_sc, p['W5'], p['W6'].T, p['W7'].T, p['W8'].T,
        p['W9'], p['W10'], dec_cst)

    vq_loss = vq_out.reshape(())
    return (zet.T, zqt.T, vq_loss, xht.T, zaugt.T, gt.T)
